# baseline (speedup 1.0000x reference)
import functools

import jax
import jax.numpy as jnp
from jax import lax
from jax.experimental import pallas as pl
from jax.experimental.pallas import tpu as pltpu
from jax.experimental.pallas import tpu_sc as plsc

_B, _L, _E = 1024, 200, 64
_N = _B * _L
_MAXLEN = 512
_CH = 128


@functools.cache
def _make_sc_kernel():
    info = plsc.get_sparse_core_info()
    nc, ns = info.num_cores, info.num_subcores
    nw = nc * ns
    pw = _N // nw
    nch = pw // _CH
    mesh = plsc.VectorSubcoreMesh(core_axis_name="c", subcore_axis_name="s",
                                  num_cores=nc)

    @functools.partial(
        pl.kernel,
        mesh=mesh,
        compiler_params=pltpu.CompilerParams(needs_layout_passes=False,
                                             use_tc_tiling_on_sc=False),
        out_type=jax.ShapeDtypeStruct((_N, _E), jnp.float32),
        scratch_types=[
            pltpu.VMEM((_L * _E,), jnp.float32),
            pltpu.VMEM((2 * _E,), jnp.float32),
            pltpu.VMEM((2 * _L * _E,), jnp.float32),
            pltpu.VMEM((_CH,), jnp.int32),
            pltpu.VMEM((_CH,), jnp.int32),
            pltpu.VMEM((_CH, _E), jnp.float32),
            pltpu.SemaphoreType.DMA,
        ],
    )
    def sc_kernel(ids_hbm, segids_hbm, tok_hbm, pos_hbm, seg_hbm,
                  out_hbm, pos_v, seg_v, comb_v, idx_v, segc_v, rows_v, sem):
        wid = lax.axis_index("s") * nc + lax.axis_index("c")
        base = wid * pw

        pltpu.sync_copy(pos_hbm.at[pl.ds(0, _L * _E)], pos_v)
        pltpu.sync_copy(seg_hbm.at[pl.ds(0, 2 * _E)], seg_v)

        def build(l, carry):
            for j in range(_E // 16):
                p = pos_v[pl.ds(l * _E + 16 * j, 16)]
                s0 = seg_v[pl.ds(16 * j, 16)]
                s1 = seg_v[pl.ds(_E + 16 * j, 16)]
                comb_v[pl.ds(2 * l * _E + 16 * j, 16)] = p + s0
                comb_v[pl.ds((2 * l + 1) * _E + 16 * j, 16)] = p + s1
            return carry

        lax.fori_loop(0, _L, build, 0)

        lane = lax.iota(jnp.int32, 16)
        zeros = lane - lane

        def chunk_body(ch, carry):
            rowbase = base + ch * _CH
            pltpu.sync_copy(ids_hbm.at[pl.ds(rowbase, _CH)], idx_v)
            pltpu.sync_copy(segids_hbm.at[pl.ds(rowbase, _CH)], segc_v)
            pltpu.async_copy(tok_hbm.at[idx_v], rows_v, sem).wait()
            for g in range(_CH // 16):
                row_vec = g * 16 + lane
                l_vec = lax.rem(rowbase + row_vec, _L)
                seg_vec = segc_v[pl.ds(g * 16, 16)]
                cstart = (l_vec * 2 + seg_vec) * _E

                def col_body(c, carry2):
                    comb_vec, col_vec = carry2
                    a = plsc.load_gather(comb_v, [comb_vec])
                    plsc.addupdate_scatter(rows_v, [row_vec, col_vec], a)
                    return (comb_vec + 1, col_vec + 1)

                lax.fori_loop(0, _E, col_body, (cstart, zeros))
            pltpu.sync_copy(rows_v, out_hbm.at[pl.ds(rowbase, _CH)])
            return carry

        lax.fori_loop(0, nch, chunk_body, 0)

    return sc_kernel


def kernel(input_ids, segment_ids, token_table, pos_table, seg_table):
    ids_flat = input_ids.reshape(_N)
    seg_flat = segment_ids.reshape(_N)
    pos_flat = pos_table.reshape(_MAXLEN * _E)
    segtab_flat = seg_table.reshape(2 * _E)
    out = _make_sc_kernel()(ids_flat, seg_flat, token_table, pos_flat,
                            segtab_flat)
    return out.reshape(_B, _L, _E)


# speedup vs baseline: 1.0739x; 1.0739x over previous
import functools

import jax
import jax.numpy as jnp
from jax import lax
from jax.experimental import pallas as pl
from jax.experimental.pallas import tpu as pltpu
from jax.experimental.pallas import tpu_sc as plsc

_B, _L, _E = 1024, 200, 64
_MAXLEN = 512
_BPC = 2
_CR = _BPC * _L
_SPLITS = ((0, 128), (128, 72), (200, 128), (328, 72))


@functools.cache
def _make_sc_kernel():
    info = plsc.get_sparse_core_info()
    nc, ns = info.num_cores, info.num_subcores
    nw = nc * ns
    bpw = _B // nw
    nch = bpw // _BPC
    mesh = plsc.VectorSubcoreMesh(core_axis_name="c", subcore_axis_name="s",
                                  num_cores=nc)

    @functools.partial(
        pl.kernel,
        mesh=mesh,
        compiler_params=pltpu.CompilerParams(needs_layout_passes=False,
                                             use_tc_tiling_on_sc=False),
        out_type=jax.ShapeDtypeStruct((_B, _L, _E), jnp.float32),
        scratch_types=[
            pltpu.VMEM((_L, _E), jnp.float32),
            pltpu.VMEM((2, _E), jnp.float32),
            pltpu.VMEM((2 * _L * _E,), jnp.float32),
            pltpu.VMEM((_CR,), jnp.int32),
            pltpu.VMEM((_CR,), jnp.int32),
            pltpu.VMEM((_CR,), jnp.int32),
            pltpu.VMEM((_CR,), jnp.int32),
            pltpu.VMEM((_CR, _E), jnp.float32),
            pltpu.VMEM((_CR, _E), jnp.float32),
            pltpu.SemaphoreType.DMA,
            pltpu.SemaphoreType.DMA,
            pltpu.SemaphoreType.DMA,
            pltpu.SemaphoreType.DMA,
            pltpu.SemaphoreType.DMA,
            pltpu.SemaphoreType.DMA,
            pltpu.SemaphoreType.DMA,
            pltpu.SemaphoreType.DMA,
        ],
    )
    def sc_kernel(ids_hbm, segids_hbm, tok_hbm, pos_hbm, seg_hbm, out_hbm,
                  pos_v, seg_v, comb_v, idx0, idx1, sgc0, sgc1, rows0, rows1,
                  sa0, sa1, ss0, ss1, sg0, sg1, so0, so1):
        idx_b, sgc_b, rows_b = (idx0, idx1), (sgc0, sgc1), (rows0, rows1)
        sem_a, sem_s, sem_g, sem_o = (sa0, sa1), (ss0, ss1), (sg0, sg1), (so0, so1)

        wid = lax.axis_index("s") * nc + lax.axis_index("c")
        bbase = wid * bpw

        pltpu.sync_copy(pos_hbm.at[pl.ds(0, _L)], pos_v)
        pltpu.sync_copy(seg_hbm.at[pl.ds(0, 2)], seg_v)

        def build(l, carry):
            for j in range(_E // 16):
                p = pos_v[l, pl.ds(16 * j, 16)]
                s0 = seg_v[0, pl.ds(16 * j, 16)]
                s1 = seg_v[1, pl.ds(16 * j, 16)]
                comb_v[pl.ds(2 * l * _E + 16 * j, 16)] = p + s0
                comb_v[pl.ds((2 * l + 1) * _E + 16 * j, 16)] = p + s1
            return carry

        lax.fori_loop(0, _L, build, 0, unroll=False)

        def enq_rowpair(src_hbm, dst, k, sem):
            b = bbase + _BPC * k
            for i, (off, n) in enumerate(_SPLITS):
                pltpu.async_copy(src_hbm.at[b + off // _L, pl.ds(off % _L, n)],
                                 dst.at[pl.ds(off, n)], sem)

        def wait_rowpair(src_hbm, dst, sem):
            for off, n in _SPLITS:
                pltpu.make_async_copy(src_hbm.at[0, pl.ds(0, n)],
                                      dst.at[pl.ds(off, n)], sem).wait()

        def enq_gather(p):
            for off, n in _SPLITS:
                pltpu.async_copy(tok_hbm.at[idx_b[p].at[pl.ds(off, n)]],
                                 rows_b[p].at[pl.ds(off, n)], sem_g[p])

        def wait_gather(p):
            for off, n in _SPLITS:
                pltpu.make_async_copy(tok_hbm.at[idx_b[p].at[pl.ds(off, n)]],
                                      rows_b[p].at[pl.ds(off, n)],
                                      sem_g[p]).wait()

        def enq_out(k, p):
            b = bbase + _BPC * k
            for i in range(_BPC):
                pltpu.async_copy(rows_b[p].at[pl.ds(i * _L, _L)],
                                 out_hbm.at[b + i], sem_o[p])

        def wait_out(p):
            for i in range(_BPC):
                pltpu.make_async_copy(rows_b[p].at[pl.ds(i * _L, _L)],
                                      out_hbm.at[0], sem_o[p]).wait()

        lane = lax.iota(jnp.int32, 16)

        def compute(p):
            rows, sgc = rows_b[p], sgc_b[p]

            def group(g, carry):
                row0 = g * 16
                row_vec = row0 + lane
                l_vec = lax.rem(row_vec, _L)
                seg_vec = sgc[pl.ds(row0, 16)]
                cvec0 = (l_vec * 2 + seg_vec) * _E

                def cols(c, carry2):
                    cvec, colvec = carry2
                    for u in range(4):
                        a = plsc.load_gather(comb_v, [cvec + u])
                        plsc.addupdate_scatter(rows, [row_vec, colvec + u], a)
                    return (cvec + 4, colvec + 4)

                lax.fori_loop(0, _E // 4, cols, (cvec0, lane - lane),
                              unroll=False)
                return carry

            lax.fori_loop(0, _CR // 16, group, 0, unroll=False)

        enq_rowpair(ids_hbm, idx_b[0], 0, sem_a[0])
        enq_rowpair(segids_hbm, sgc_b[0], 0, sem_s[0])
        enq_rowpair(ids_hbm, idx_b[1], 1, sem_a[1])
        enq_rowpair(segids_hbm, sgc_b[1], 1, sem_s[1])
        wait_rowpair(ids_hbm, idx_b[0], sem_a[0])
        enq_gather(0)

        def pair(i, carry):
            for p in (0, 1):
                k = 2 * i + p
                q = 1 - p
                wait_gather(p)

                @pl.when(i < nch // 2 - 1)
                def _():
                    enq_rowpair(ids_hbm, idx_b[p], k + 2, sem_a[p])

                def start_next():
                    wait_rowpair(ids_hbm, idx_b[q], sem_a[q])
                    enq_gather(q)

                if p == 0:
                    @pl.when(i >= 1)
                    def _():
                        wait_out(q)
                    start_next()
                else:
                    @pl.when(i < nch // 2 - 1)
                    def _():
                        wait_out(q)
                        start_next()

                wait_rowpair(segids_hbm, sgc_b[p], sem_s[p])
                compute(p)
                enq_out(k, p)

                @pl.when(i < nch // 2 - 1)
                def _():
                    enq_rowpair(segids_hbm, sgc_b[p], k + 2, sem_s[p])
            return carry

        lax.fori_loop(0, nch // 2, pair, 0, unroll=False)
        wait_out(0)
        wait_out(1)

    return sc_kernel


def kernel(input_ids, segment_ids, token_table, pos_table, seg_table):
    return _make_sc_kernel()(input_ids, segment_ids, token_table, pos_table,
                             seg_table)
